# baseline (speedup 1.0000x reference)
import functools

import jax
import jax.numpy as jnp
from jax import lax
from jax.experimental import pallas as pl
from jax.experimental.pallas import tpu as pltpu
from jax.experimental.pallas import tpu_sc as plsc

N = 20000
NP = 20480
MAX_DET = 100
CONF_THRES = 0.2
IOU_THRES = 0.4
NEG_INF = float("-inf")

L = 16
NTILES = 16
PER = NP // NTILES
NV = PER // L


def _lane_iota():
    return lax.broadcasted_iota(jnp.int32, (L,), 0)


def _splat(x, dtype=jnp.float32):
    return jnp.broadcast_to(jnp.asarray(x, dtype), (L,))


def _nms_sc_body(x1_hbm, y1_hbm, x2_hbm, y2_hbm, s_hbm, out_hbm,
                 x1_v, y1_v, x2_v, y2_v, ms_v, rec_v, cand_v, out_v,
                 shared):
    wid = lax.axis_index("s")
    base = wid * PER
    lane = _lane_iota()

    pltpu.sync_copy(x1_hbm.at[pl.ds(base, PER)], x1_v)
    pltpu.sync_copy(y1_hbm.at[pl.ds(base, PER)], y1_v)
    pltpu.sync_copy(x2_hbm.at[pl.ds(base, PER)], x2_v)
    pltpu.sync_copy(y2_hbm.at[pl.ds(base, PER)], y2_v)
    pltpu.sync_copy(s_hbm.at[pl.ds(base, PER)], ms_v)

    def maxpass(k, acc):
        return jnp.maximum(acc, ms_v[pl.ds(k * L, L)])

    smax_vec = lax.fori_loop(0, NV, maxpass, _splat(NEG_INF))
    smax = jnp.max(smax_vec)
    rec_v[...] = jnp.broadcast_to(smax, (L,))
    pltpu.sync_copy(rec_v, shared.at[pl.ds(wid * L, L)])
    plsc.subcore_barrier()
    pltpu.sync_copy(shared, cand_v)
    plsc.subcore_barrier()
    col0 = plsc.load_gather(cand_v, [lane * L])
    any_above = jnp.max(col0) > CONF_THRES

    def actpass(k, _):
        sl = pl.ds(k * L, L)
        sv = ms_v[sl]
        dw = x2_v[sl] - x1_v[sl]
        dh = y2_v[sl] - y1_v[sl]
        act = ((sv > CONF_THRES) | jnp.logical_not(any_above)) \
            & (dw >= 1.0) & (dh >= 1.0)
        ms_v[sl] = jnp.where(act, sv, NEG_INF)
        return 0

    lax.fori_loop(0, NV, actpass, 0)

    @pl.when(wid == 0)
    def _():
        def zpass(k, _):
            out_v[pl.ds(k * L, L)] = jnp.zeros((L,), jnp.float32)
            return 0
        lax.fori_loop(0, 64, zpass, 0)

    def round_step(i, box):
        bx1, by1, bx2, by2 = box
        barea = (bx2 - bx1) * (by2 - by1)

        init = (_splat(NEG_INF), jnp.full((L,), 2**24, jnp.int32))

        @plsc.parallel_loop(0, NV, carry=init, unroll=4)
        def fused(k, carry):
            mx, mi = carry
            sl = pl.ds(k * L, L)
            x1 = x1_v[sl]
            y1 = y1_v[sl]
            x2 = x2_v[sl]
            y2 = y2_v[sl]
            area = (x2 - x1) * (y2 - y1)
            ix1 = jnp.maximum(bx1, x1)
            iy1 = jnp.maximum(by1, y1)
            ix2 = jnp.minimum(bx2, x2)
            iy2 = jnp.minimum(by2, y2)
            inter = jnp.clip(ix2 - ix1, 0.0) * jnp.clip(iy2 - iy1, 0.0)
            iou = inter / (barea + area - inter + 1e-9)
            msv = jnp.where(iou > IOU_THRES, NEG_INF, ms_v[sl])
            ms_v[sl] = msv
            kv = jnp.broadcast_to(k, (L,))
            take = (msv > mx) | ((msv == mx) & (kv < mi))
            mx = jnp.where(take, msv, mx)
            mi = jnp.where(take, kv, mi)
            return mx, mi

        mx, mi = fused

        m_loc = jnp.max(mx)
        j_loc = jnp.min(jnp.where(mx == m_loc, base + mi * L + lane,
                                  jnp.int32(2**30)))
        lidx = jnp.broadcast_to(jnp.clip(j_loc - base, 0, PER - 1), (L,))
        cx1 = plsc.load_gather(x1_v, [lidx])
        cy1 = plsc.load_gather(y1_v, [lidx])
        cx2 = plsc.load_gather(x2_v, [lidx])
        cy2 = plsc.load_gather(y2_v, [lidx])

        rec = jnp.where(lane == 0, jnp.broadcast_to(m_loc, (L,)),
              jnp.where(lane == 1, jnp.broadcast_to(
                  j_loc.astype(jnp.float32), (L,)),
              jnp.where(lane == 2, cx1,
              jnp.where(lane == 3, cy1,
              jnp.where(lane == 4, cx2, cy2)))))
        rec_v[...] = rec
        pltpu.sync_copy(rec_v, shared.at[pl.ds(wid * L, L)])
        plsc.subcore_barrier()
        pltpu.sync_copy(shared, cand_v)
        plsc.subcore_barrier()

        def col(c):
            return plsc.load_gather(cand_v, [lane * L + c])

        mvec = col(0)
        jvec = col(1)
        m_g = jnp.max(mvec)
        valid = m_g > NEG_INF
        eq = mvec == m_g
        j_g = jnp.min(jnp.where(eq, jvec, jnp.float32(1e30)))
        j_gi = jnp.clip(j_g, 0.0, float(NP - 1)).astype(jnp.int32)
        wbase = (j_gi // PER) * L
        wx1 = plsc.load_gather(cand_v, [jnp.broadcast_to(wbase + 2, (L,))])
        wy1 = plsc.load_gather(cand_v, [jnp.broadcast_to(wbase + 3, (L,))])
        wx2 = plsc.load_gather(cand_v, [jnp.broadcast_to(wbase + 4, (L,))])
        wy2 = plsc.load_gather(cand_v, [jnp.broadcast_to(wbase + 5, (L,))])

        @pl.when(wid == 0)
        def _():
            vf = jnp.where(valid, jnp.float32(1.0), jnp.float32(0.0))
            msk = lane == 0
            vals = (wx1 * vf, wy1 * vf, wx2 * vf, wy2 * vf,
                    jnp.broadcast_to(
                        jnp.where(valid, m_g, jnp.float32(0.0)), (L,)), vf)
            for r, val in enumerate(vals):
                idx = [jnp.broadcast_to(r * 128 + i, (L,))]
                plsc.store_scatter(
                    out_v, idx, jnp.broadcast_to(val, (L,)), mask=msk)

        return wx1, wy1, wx2, wy2

    z = jnp.zeros((L,), jnp.float32)
    lax.fori_loop(0, MAX_DET, round_step, (z, z, z, z))

    @pl.when(wid == 0)
    def _():
        pltpu.sync_copy(out_v, out_hbm)


@functools.partial(jax.jit, static_argnames=("interpret",))
def kernel(boxes, scores, interpret=False):
    boxes_p = jnp.pad(boxes, ((0, NP - N), (0, 0)))
    scores_p = jnp.pad(scores, (0, NP - N))
    x1 = boxes_p[:, 0]
    y1 = boxes_p[:, 1]
    x2 = boxes_p[:, 2]
    y2 = boxes_p[:, 3]

    mesh = plsc.VectorSubcoreMesh(
        core_axis_name="c", subcore_axis_name="s", num_cores=1,
        num_subcores=NTILES)
    f32 = jnp.float32
    out = pl.kernel(
        _nms_sc_body,
        out_type=jax.ShapeDtypeStruct((1024,), f32),
        mesh=mesh,
        scratch_types=[
            pltpu.VMEM((PER,), f32),
            pltpu.VMEM((PER,), f32),
            pltpu.VMEM((PER,), f32),
            pltpu.VMEM((PER,), f32),
            pltpu.VMEM((PER,), f32),
            pltpu.VMEM((L,), f32),
            pltpu.VMEM((NTILES * L,), f32),
            pltpu.VMEM((1024,), f32),
            pltpu.VMEM_SHARED((NTILES * L,), f32),
        ],
        compiler_params=pltpu.CompilerParams(needs_layout_passes=False),
        interpret=interpret,
    )(x1, y1, x2, y2, scores_p)
    out = out.reshape(8, 128)

    kept_boxes = jnp.stack(
        [out[0, :MAX_DET], out[1, :MAX_DET], out[2, :MAX_DET], out[3, :MAX_DET]],
        axis=1)
    kept_scores = out[4, :MAX_DET]
    selmask = out[5, :MAX_DET] > 0.5
    return kept_boxes, kept_scores, selmask


# speedup vs baseline: 1.1174x; 1.1174x over previous
import functools

import jax
import jax.numpy as jnp
from jax import lax
from jax.experimental import pallas as pl
from jax.experimental.pallas import tpu as pltpu
from jax.experimental.pallas import tpu_sc as plsc

N = 20000
NP = 20480
MAX_DET = 100
CONF_THRES = 0.2
IOU_THRES = 0.4
NEG_INF = float("-inf")

L = 16
NTILES = 16
PER = NP // NTILES
NV = PER // L


def _lane_iota():
    return lax.broadcasted_iota(jnp.int32, (L,), 0)


def _splat(x, dtype=jnp.float32):
    return jnp.broadcast_to(jnp.asarray(x, dtype), (L,))


def _nms_sc_body(x1_hbm, y1_hbm, x2_hbm, y2_hbm, s_hbm, out_hbm,
                 x1_v, y1_v, x2_v, y2_v, ms_v, rec_v, cand_v, out_v,
                 shared, shared_b):
    wid = lax.axis_index("s")
    base = wid * PER
    lane = _lane_iota()

    pltpu.sync_copy(x1_hbm.at[pl.ds(base, PER)], x1_v)
    pltpu.sync_copy(y1_hbm.at[pl.ds(base, PER)], y1_v)
    pltpu.sync_copy(x2_hbm.at[pl.ds(base, PER)], x2_v)
    pltpu.sync_copy(y2_hbm.at[pl.ds(base, PER)], y2_v)
    pltpu.sync_copy(s_hbm.at[pl.ds(base, PER)], ms_v)

    def maxpass(k, acc):
        return jnp.maximum(acc, ms_v[pl.ds(k * L, L)])

    smax_vec = lax.fori_loop(0, NV, maxpass, _splat(NEG_INF))
    smax = jnp.max(smax_vec)
    rec_v[...] = jnp.broadcast_to(smax, (L,))
    pltpu.sync_copy(rec_v, shared.at[pl.ds(wid * L, L)])
    plsc.subcore_barrier()
    pltpu.sync_copy(shared, cand_v)
    plsc.subcore_barrier()
    col0 = plsc.load_gather(cand_v, [lane * L])
    any_above = jnp.max(col0) > CONF_THRES

    def actpass(k, _):
        sl = pl.ds(k * L, L)
        sv = ms_v[sl]
        dw = x2_v[sl] - x1_v[sl]
        dh = y2_v[sl] - y1_v[sl]
        act = ((sv > CONF_THRES) | jnp.logical_not(any_above)) \
            & (dw >= 1.0) & (dh >= 1.0)
        ms_v[sl] = jnp.where(act, sv, NEG_INF)
        return 0

    lax.fori_loop(0, NV, actpass, 0)

    @pl.when(wid == 0)
    def _():
        def zpass(k, _):
            out_v[pl.ds(k * L, L)] = jnp.zeros((L,), jnp.float32)
            return 0
        lax.fori_loop(0, 64, zpass, 0)

    def round_step(i, box):
        bx1, by1, bx2, by2 = box
        barea = (bx2 - bx1) * (by2 - by1)

        UNR = 4
        def fused(k, carry):
            new = []
            for j in range(UNR):
                mx, mi = carry[2 * j], carry[2 * j + 1]
                kg = k * UNR + j
                sl = pl.ds(kg * L, L)
                x1 = x1_v[sl]
                y1 = y1_v[sl]
                x2 = x2_v[sl]
                y2 = y2_v[sl]
                area = (x2 - x1) * (y2 - y1)
                ix1 = jnp.maximum(bx1, x1)
                iy1 = jnp.maximum(by1, y1)
                ix2 = jnp.minimum(bx2, x2)
                iy2 = jnp.minimum(by2, y2)
                inter = jnp.clip(ix2 - ix1, 0.0) * jnp.clip(iy2 - iy1, 0.0)
                iou = inter / (barea + area - inter + 1e-9)
                msv = jnp.where(iou > IOU_THRES, NEG_INF, ms_v[sl])
                ms_v[sl] = msv
                take = msv > mx
                mx = jnp.where(take, msv, mx)
                mi = jnp.where(take, jnp.broadcast_to(kg, (L,)), mi)
                new += [mx, mi]
            return tuple(new)

        init = (_splat(NEG_INF), jnp.zeros((L,), jnp.int32)) * UNR
        acc = lax.fori_loop(0, NV // UNR, fused, init)
        mx, gi = acc[0], acc[1] * L + lane
        for j in range(1, UNR):
            mxj, gij = acc[2 * j], acc[2 * j + 1] * L + lane
            takej = (mxj > mx) | ((mxj == mx) & (gij < gi))
            mx = jnp.where(takej, mxj, mx)
            gi = jnp.where(takej, gij, gi)

        m_loc = jnp.max(mx)
        j_loc = jnp.min(jnp.where(mx == m_loc, base + gi, jnp.int32(2**30)))
        lidx = jnp.broadcast_to(jnp.clip(j_loc - base, 0, PER - 1), (L,))
        cx1 = plsc.load_gather(x1_v, [lidx])
        cy1 = plsc.load_gather(y1_v, [lidx])
        cx2 = plsc.load_gather(x2_v, [lidx])
        cy2 = plsc.load_gather(y2_v, [lidx])

        rec = jnp.where(lane == 0, jnp.broadcast_to(m_loc, (L,)),
              jnp.where(lane == 1, jnp.broadcast_to(
                  j_loc.astype(jnp.float32), (L,)),
              jnp.where(lane == 2, cx1,
              jnp.where(lane == 3, cy1,
              jnp.where(lane == 4, cx2, cy2)))))
        rec_v[...] = rec

        @pl.when(i % 2 == 0)
        def _():
            pltpu.sync_copy(rec_v, shared.at[pl.ds(wid * L, L)])
            plsc.subcore_barrier()
            pltpu.sync_copy(shared, cand_v)

        @pl.when(i % 2 == 1)
        def _():
            pltpu.sync_copy(rec_v, shared_b.at[pl.ds(wid * L, L)])
            plsc.subcore_barrier()
            pltpu.sync_copy(shared_b, cand_v)

        def col(c):
            return plsc.load_gather(cand_v, [lane * L + c])

        mvec = col(0)
        jvec = col(1)
        m_g = jnp.max(mvec)
        valid = m_g > NEG_INF
        eq = mvec == m_g
        j_g = jnp.min(jnp.where(eq, jvec, jnp.float32(1e30)))
        j_gi = jnp.clip(j_g, 0.0, float(NP - 1)).astype(jnp.int32)
        wbase = (j_gi // PER) * L
        wx1 = plsc.load_gather(cand_v, [jnp.broadcast_to(wbase + 2, (L,))])
        wy1 = plsc.load_gather(cand_v, [jnp.broadcast_to(wbase + 3, (L,))])
        wx2 = plsc.load_gather(cand_v, [jnp.broadcast_to(wbase + 4, (L,))])
        wy2 = plsc.load_gather(cand_v, [jnp.broadcast_to(wbase + 5, (L,))])

        @pl.when(wid == 0)
        def _():
            vf = jnp.where(valid, jnp.float32(1.0), jnp.float32(0.0))
            msk = lane == 0
            vals = (wx1 * vf, wy1 * vf, wx2 * vf, wy2 * vf,
                    jnp.broadcast_to(
                        jnp.where(valid, m_g, jnp.float32(0.0)), (L,)), vf)
            for r, val in enumerate(vals):
                idx = [jnp.broadcast_to(r * 128 + i, (L,))]
                plsc.store_scatter(
                    out_v, idx, jnp.broadcast_to(val, (L,)), mask=msk)

        return wx1, wy1, wx2, wy2

    z = jnp.zeros((L,), jnp.float32)
    lax.fori_loop(0, MAX_DET, round_step, (z, z, z, z))

    @pl.when(wid == 0)
    def _():
        pltpu.sync_copy(out_v, out_hbm)


@functools.partial(jax.jit, static_argnames=("interpret",))
def kernel(boxes, scores, interpret=False):
    boxes_p = jnp.pad(boxes, ((0, NP - N), (0, 0)))
    scores_p = jnp.pad(scores, (0, NP - N))
    x1 = boxes_p[:, 0]
    y1 = boxes_p[:, 1]
    x2 = boxes_p[:, 2]
    y2 = boxes_p[:, 3]

    mesh = plsc.VectorSubcoreMesh(
        core_axis_name="c", subcore_axis_name="s", num_cores=1,
        num_subcores=NTILES)
    f32 = jnp.float32
    out = pl.kernel(
        _nms_sc_body,
        out_type=jax.ShapeDtypeStruct((1024,), f32),
        mesh=mesh,
        scratch_types=[
            pltpu.VMEM((PER,), f32),
            pltpu.VMEM((PER,), f32),
            pltpu.VMEM((PER,), f32),
            pltpu.VMEM((PER,), f32),
            pltpu.VMEM((PER,), f32),
            pltpu.VMEM((L,), f32),
            pltpu.VMEM((NTILES * L,), f32),
            pltpu.VMEM((1024,), f32),
            pltpu.VMEM_SHARED((NTILES * L,), f32),
            pltpu.VMEM_SHARED((NTILES * L,), f32),
        ],
        compiler_params=pltpu.CompilerParams(needs_layout_passes=False),
        interpret=interpret,
    )(x1, y1, x2, y2, scores_p)
    out = out.reshape(8, 128)

    kept_boxes = jnp.stack(
        [out[0, :MAX_DET], out[1, :MAX_DET], out[2, :MAX_DET], out[3, :MAX_DET]],
        axis=1)
    kept_scores = out[4, :MAX_DET]
    selmask = out[5, :MAX_DET] > 0.5
    return kept_boxes, kept_scores, selmask
